# Initial kernel scaffold; baseline (speedup 1.0000x reference)
import jax
import jax.numpy as jnp
from jax.experimental import pallas as pl


def kernel(x, A_ind, A_val):
    raise NotImplementedError("write your pallas kernel here")


# speedup vs baseline: 4.5569x; 4.5569x over previous
import functools

import jax
import jax.numpy as jnp
from jax import lax
from jax.experimental import pallas as pl
from jax.experimental.pallas import tpu as pltpu
from jax.experimental.pallas import tpu_sc as plsc

N = 10000
NP = 10240
D = 128
NC = 2
NS = 16
NW = NC * NS
B = 128
ROWS_PER_TILE = NP // NS
ROW_CHUNK = 128


def _bcast_lane(v16, lane):
    idx = jnp.full((16, 1), lane, dtype=jnp.int32)
    dn = lax.GatherDimensionNumbers(
        offset_dims=(), collapsed_slice_dims=(0,), start_index_map=(0,))
    return lax.gather(v16, idx, dn, slice_sizes=(1,),
                      mode=lax.GatherScatterMode.PROMISE_IN_BOUNDS)


def _sc_spmv_partial(x, col, row, val, nb_per_tile):
    mesh = plsc.VectorSubcoreMesh(core_axis_name="c", subcore_axis_name="s")

    def body(x_hbm, col_hbm, row_hbm, val_hbm, out_hbm,
             colbuf, rowbuf, valbuf, rowsbuf, acc_sh, sem):
        cid = lax.axis_index("c")
        sid = lax.axis_index("s")
        wid = sid * NC + cid

        def zbody(i, carry):
            for k in range(D // 16):
                rowsbuf[i, pl.ds(k * 16, 16)] = jnp.zeros((16,), jnp.float32)
            return carry
        lax.fori_loop(0, B, zbody, 0)
        for c in range(ROWS_PER_TILE // ROW_CHUNK):
            pltpu.sync_copy(
                rowsbuf,
                acc_sh.at[pl.ds(sid * ROWS_PER_TILE + c * ROW_CHUNK, ROW_CHUNK)])
        plsc.subcore_barrier()

        pltpu.sync_copy(col_hbm.at[wid], colbuf)
        pltpu.sync_copy(row_hbm.at[wid], rowbuf)
        pltpu.sync_copy(val_hbm.at[wid], valbuf)

        def batch(nb, carry):
            pltpu.async_copy(x_hbm.at[colbuf.at[nb]], rowsbuf, sem).wait()

            def jbody(j, c2):
                v16 = valbuf[nb, pl.ds(j * 16, 16)]
                for e in range(16):
                    bv = _bcast_lane(v16, e)
                    r = j * 16 + e
                    for k in range(D // 16):
                        rowsbuf[r, pl.ds(k * 16, 16)] = (
                            rowsbuf[r, pl.ds(k * 16, 16)] * bv)
                return c2
            lax.fori_loop(0, B // 16, jbody, 0)

            pltpu.sync_copy(rowsbuf, acc_sh.at[rowbuf.at[nb]], add=True)
            return carry
        lax.fori_loop(0, nb_per_tile, batch, 0)

        plsc.subcore_barrier()

        for c in range(ROWS_PER_TILE // ROW_CHUNK):
            off = sid * ROWS_PER_TILE + c * ROW_CHUNK
            pltpu.sync_copy(acc_sh.at[pl.ds(off, ROW_CHUNK)],
                            out_hbm.at[cid, pl.ds(off, ROW_CHUNK)])

    run = pl.kernel(
        body,
        mesh=mesh,
        out_type=jax.ShapeDtypeStruct((NC, NP, D), jnp.float32),
        scratch_types=[
            pltpu.VMEM((nb_per_tile, B), jnp.int32),
            pltpu.VMEM((nb_per_tile, B), jnp.int32),
            pltpu.VMEM((nb_per_tile, B), jnp.float32),
            pltpu.VMEM((B, D), jnp.float32),
            pltpu.VMEM_SHARED((NP, D), jnp.float32),
            pltpu.SemaphoreType.DMA,
        ],
    )
    return run(x, col, row, val)


def _tc_add(partial):
    BLK = 1024

    def body(p_ref, o_ref):
        o_ref[...] = p_ref[0] + p_ref[1]

    return pl.pallas_call(
        body,
        out_shape=jax.ShapeDtypeStruct((NP, D), jnp.float32),
        grid=(NP // BLK,),
        in_specs=[pl.BlockSpec((NC, BLK, D), lambda i: (0, i, 0))],
        out_specs=pl.BlockSpec((BLK, D), lambda i: (i, 0)),
    )(partial)


@jax.jit
def kernel(x, A_ind, A_val):
    row = A_ind[0].astype(jnp.int32)
    col = A_ind[1].astype(jnp.int32)
    val = A_val.astype(jnp.float32)
    e = val.shape[0]
    chunk = NW * B
    e_pad = ((e + chunk - 1) // chunk) * chunk
    if e_pad != e:
        pad = e_pad - e
        row = jnp.concatenate([row, jnp.zeros((pad,), jnp.int32)])
        col = jnp.concatenate([col, jnp.zeros((pad,), jnp.int32)])
        val = jnp.concatenate([val, jnp.zeros((pad,), jnp.float32)])
    nb_per_tile = e_pad // chunk
    col3 = col.reshape(NW, nb_per_tile, B)
    row3 = row.reshape(NW, nb_per_tile, B)
    val3 = val.reshape(NW, nb_per_tile, B)
    partial = _sc_spmv_partial(x, col3, row3, val3, nb_per_tile)
    return _tc_add(partial)[:N]


# speedup vs baseline: 5.3885x; 1.1825x over previous
import functools

import jax
import jax.numpy as jnp
from jax import lax
from jax.experimental import pallas as pl
from jax.experimental.pallas import tpu as pltpu
from jax.experimental.pallas import tpu_sc as plsc

N = 10000
NP = 10240
D = 128
DH = D // 2
NC = 2
NS = 16
B = 128
PASS = 80
NPASS = 2
NB = PASS * NPASS
ROWS_PER_TILE = NP // NS


def _bcast_lane(v16, lane):
    idx = jnp.full((16, 1), lane, dtype=jnp.int32)
    dn = lax.GatherDimensionNumbers(
        offset_dims=(), collapsed_slice_dims=(0,), start_index_map=(0,))
    return lax.gather(v16, idx, dn, slice_sizes=(1,),
                      mode=lax.GatherScatterMode.PROMISE_IN_BOUNDS)


def _sc_spmv_partial(xh, col, row, val):
    mesh = plsc.VectorSubcoreMesh(core_axis_name="c", subcore_axis_name="s")

    def body(x_hbm, col_hbm, row_hbm, val_hbm, out_hbm,
             colbuf, rowbuf, valbuf,
             gbuf0, gbuf1, sbuf0, sbuf1, acc_sh,
             gsem0, gsem1, ssem0, ssem1):
        gbufs = (gbuf0, gbuf1)
        sbufs = (sbuf0, sbuf1)
        gsems = (gsem0, gsem1)
        ssems = (ssem0, ssem1)
        cid = lax.axis_index("c")
        sid = lax.axis_index("s")
        xc = x_hbm.at[cid]

        def zbody(i, carry):
            for k in range(DH // 16):
                sbuf0[i, pl.ds(k * 16, 16)] = jnp.zeros((16,), jnp.float32)
            return carry
        lax.fori_loop(0, B, zbody, 0)
        for c in range(ROWS_PER_TILE // B):
            pltpu.sync_copy(
                sbuf0, acc_sh.at[pl.ds(sid * ROWS_PER_TILE + c * B, B)])
        plsc.subcore_barrier()

        for ps in range(NPASS):
            sl = pl.ds(ps * PASS, PASS)
            pltpu.sync_copy(col_hbm.at[sid, sl], colbuf)
            pltpu.sync_copy(row_hbm.at[sid, sl], rowbuf)
            pltpu.sync_copy(val_hbm.at[sid, sl], valbuf)

            for p in range(2):
                pltpu.async_copy(xc.at[colbuf.at[p]], gbufs[p], gsems[p])

            def outer(t, carry):
                for p in range(2):
                    i = 2 * t + p
                    gb, sb = gbufs[p], sbufs[p]
                    pltpu.make_async_copy(xc.at[colbuf.at[i]], gb,
                                          gsems[p]).wait()

                    @pl.when(jnp.logical_or(t > 0, ps > 0))
                    def _():
                        pltpu.make_async_copy(
                            sb, acc_sh.at[rowbuf.at[i]], ssems[p]).wait()

                    def jbody(j, c2):
                        v16 = valbuf[i, pl.ds(j * 16, 16)]
                        for e in range(16):
                            bv = _bcast_lane(v16, e)
                            r = j * 16 + e
                            for k in range(DH // 16):
                                sb[r, pl.ds(k * 16, 16)] = (
                                    gb[r, pl.ds(k * 16, 16)] * bv)
                        return c2
                    lax.fori_loop(0, B // 16, jbody, 0)

                    pltpu.async_copy(sb, acc_sh.at[rowbuf.at[i]], ssems[p],
                                     add=True)

                    @pl.when(t < PASS // 2 - 1)
                    def _():
                        pltpu.async_copy(xc.at[colbuf.at[i + 2]], gb,
                                         gsems[p])
                return carry
            lax.fori_loop(0, PASS // 2, outer, 0)

        for p in range(2):
            pltpu.make_async_copy(sbufs[p], acc_sh.at[rowbuf.at[PASS - 2 + p]],
                                  ssems[p]).wait()

        plsc.subcore_barrier()

        off = sid * ROWS_PER_TILE
        pltpu.sync_copy(acc_sh.at[pl.ds(off, ROWS_PER_TILE)],
                        out_hbm.at[cid, pl.ds(off, ROWS_PER_TILE)])

    run = pl.kernel(
        body,
        mesh=mesh,
        compiler_params=pltpu.CompilerParams(use_tc_tiling_on_sc=False),
        out_type=jax.ShapeDtypeStruct((NC, NP, DH), jnp.float32),
        scratch_types=[
            pltpu.VMEM((PASS, B), jnp.int32),
            pltpu.VMEM((PASS, B), jnp.int32),
            pltpu.VMEM((PASS, B), jnp.float32),
            pltpu.VMEM((B, DH), jnp.float32),
            pltpu.VMEM((B, DH), jnp.float32),
            pltpu.VMEM((B, DH), jnp.float32),
            pltpu.VMEM((B, DH), jnp.float32),
            pltpu.VMEM_SHARED((NP, DH), jnp.float32),
            pltpu.SemaphoreType.DMA,
            pltpu.SemaphoreType.DMA,
            pltpu.SemaphoreType.DMA,
            pltpu.SemaphoreType.DMA,
        ],
    )
    return run(xh, col, row, val)


def _tc_concat(partial):
    BLK = 1024

    def body(p_ref, o_ref):
        o_ref[:, :DH] = p_ref[0]
        o_ref[:, DH:] = p_ref[1]

    return pl.pallas_call(
        body,
        out_shape=jax.ShapeDtypeStruct((NP, D), jnp.float32),
        grid=(NP // BLK,),
        in_specs=[pl.BlockSpec((NC, BLK, DH), lambda i: (0, i, 0))],
        out_specs=pl.BlockSpec((BLK, D), lambda i: (i, 0)),
    )(partial)


@jax.jit
def kernel(x, A_ind, A_val):
    row = A_ind[0].astype(jnp.int32)
    col = A_ind[1].astype(jnp.int32)
    val = A_val.astype(jnp.float32)
    e = val.shape[0]
    chunk = NS * B * 2 * NPASS
    e_pad = ((e + chunk - 1) // chunk) * chunk
    if e_pad != e:
        pad = e_pad - e
        row = jnp.concatenate([row, jnp.zeros((pad,), jnp.int32)])
        col = jnp.concatenate([col, jnp.zeros((pad,), jnp.int32)])
        val = jnp.concatenate([val, jnp.zeros((pad,), jnp.float32)])
    nb = e_pad // (NS * B)
    assert nb == NB, (nb, NB)
    col3 = col.reshape(NS, NB, B)
    row3 = row.reshape(NS, NB, B)
    val3 = val.reshape(NS, NB, B)
    xh = jnp.stack([x[:, :DH], x[:, DH:]])
    partial = _sc_spmv_partial(xh, col3, row3, val3)
    return _tc_concat(partial)[:N]


# speedup vs baseline: 8.8510x; 1.6426x over previous
import jax
import jax.numpy as jnp
from jax import lax
from jax.experimental import pallas as pl
from jax.experimental.pallas import tpu as pltpu
from jax.experimental.pallas import tpu_sc as plsc

N = 10000
NP = 10240
D = 128
DH = D // 2
NC = 2
NS = 16
B = 128
PASS = 40
NPASS = 4
NB = PASS * NPASS
ROWS_PER_TILE = NP // NS


def _bcast_lane(v16, lane):
    idx = jnp.full((16, 1), lane, dtype=jnp.int32)
    dn = lax.GatherDimensionNumbers(
        offset_dims=(), collapsed_slice_dims=(0,), start_index_map=(0,))
    return lax.gather(v16, idx, dn, slice_sizes=(1,),
                      mode=lax.GatherScatterMode.PROMISE_IN_BOUNDS)


def _sc_spmv_partial(xh, col, row, val):
    mesh = plsc.VectorSubcoreMesh(core_axis_name="c", subcore_axis_name="s")

    def body(x_hbm, col_hbm, row_hbm, val_hbm, out_hbm,
             colbuf, rowbuf, valbuf,
             gbuf0, gbuf1, sbuf0, sbuf1, xsp, acc_sh,
             gsem0, gsem1, ssem0, ssem1):
        gbufs = (gbuf0, gbuf1)
        sbufs = (sbuf0, sbuf1)
        gsems = (gsem0, gsem1)
        ssems = (ssem0, ssem1)
        cid = lax.axis_index("c")
        sid = lax.axis_index("s")
        off = sid * ROWS_PER_TILE

        pltpu.sync_copy(x_hbm.at[cid, pl.ds(off, ROWS_PER_TILE)],
                        xsp.at[pl.ds(off, ROWS_PER_TILE)])

        def zbody(i, carry):
            for k in range(DH // 16):
                sbuf0[i, pl.ds(k * 16, 16)] = jnp.zeros((16,), jnp.float32)
            return carry
        lax.fori_loop(0, B, zbody, 0)
        for c in range(ROWS_PER_TILE // B):
            pltpu.sync_copy(sbuf0, acc_sh.at[pl.ds(off + c * B, B)])
        plsc.subcore_barrier()

        for ps in range(NPASS):
            sl = pl.ds(ps * PASS, PASS)
            pltpu.sync_copy(col_hbm.at[sid, sl], colbuf)
            pltpu.sync_copy(row_hbm.at[sid, sl], rowbuf)
            pltpu.sync_copy(val_hbm.at[sid, sl], valbuf)

            for p in range(2):
                pltpu.async_copy(xsp.at[colbuf.at[p]], gbufs[p], gsems[p])

            def outer(t, carry):
                for p in range(2):
                    i = 2 * t + p
                    gb, sb = gbufs[p], sbufs[p]
                    pltpu.make_async_copy(xsp.at[colbuf.at[i]], gb,
                                          gsems[p]).wait()

                    @pl.when(jnp.logical_or(t > 0, ps > 0))
                    def _():
                        pltpu.make_async_copy(
                            sb, acc_sh.at[rowbuf.at[i]], ssems[p]).wait()

                    def jbody(j, c2):
                        v16 = valbuf[i, pl.ds(j * 16, 16)]
                        for e in range(16):
                            bv = _bcast_lane(v16, e)
                            r = j * 16 + e
                            for k in range(DH // 16):
                                sb[r, pl.ds(k * 16, 16)] = (
                                    gb[r, pl.ds(k * 16, 16)] * bv)
                        return c2
                    lax.fori_loop(0, B // 16, jbody, 0)

                    pltpu.async_copy(sb, acc_sh.at[rowbuf.at[i]], ssems[p],
                                     add=True)

                    @pl.when(t < PASS // 2 - 1)
                    def _():
                        pltpu.async_copy(xsp.at[colbuf.at[i + 2]], gb,
                                         gsems[p])
                return carry
            lax.fori_loop(0, PASS // 2, outer, 0)

        for p in range(2):
            pltpu.make_async_copy(sbufs[p], acc_sh.at[rowbuf.at[PASS - 2 + p]],
                                  ssems[p]).wait()

        plsc.subcore_barrier()

        pltpu.sync_copy(acc_sh.at[pl.ds(off, ROWS_PER_TILE)],
                        out_hbm.at[cid, pl.ds(off, ROWS_PER_TILE)])

    run = pl.kernel(
        body,
        mesh=mesh,
        compiler_params=pltpu.CompilerParams(use_tc_tiling_on_sc=False),
        out_type=jax.ShapeDtypeStruct((NC, NP, DH), jnp.float32),
        scratch_types=[
            pltpu.VMEM((PASS, B), jnp.int32),
            pltpu.VMEM((PASS, B), jnp.int32),
            pltpu.VMEM((PASS, B), jnp.float32),
            pltpu.VMEM((B, DH), jnp.float32),
            pltpu.VMEM((B, DH), jnp.float32),
            pltpu.VMEM((B, DH), jnp.float32),
            pltpu.VMEM((B, DH), jnp.float32),
            pltpu.VMEM_SHARED((NP, DH), jnp.float32),
            pltpu.VMEM_SHARED((NP, DH), jnp.float32),
            pltpu.SemaphoreType.DMA,
            pltpu.SemaphoreType.DMA,
            pltpu.SemaphoreType.DMA,
            pltpu.SemaphoreType.DMA,
        ],
    )
    return run(xh, col, row, val)


def _tc_concat(partial):
    BLK = 1024

    def body(p_ref, o_ref):
        o_ref[:, :DH] = p_ref[0]
        o_ref[:, DH:] = p_ref[1]

    return pl.pallas_call(
        body,
        out_shape=jax.ShapeDtypeStruct((NP, D), jnp.float32),
        grid=(NP // BLK,),
        in_specs=[pl.BlockSpec((NC, BLK, DH), lambda i: (0, i, 0))],
        out_specs=pl.BlockSpec((BLK, D), lambda i: (i, 0)),
    )(partial)


@jax.jit
def kernel(x, A_ind, A_val):
    row = A_ind[0].astype(jnp.int32)
    col = A_ind[1].astype(jnp.int32)
    val = A_val.astype(jnp.float32)
    e = val.shape[0]
    chunk = NS * B * 2 * NPASS
    e_pad = ((e + chunk - 1) // chunk) * chunk
    if e_pad != e:
        pad = e_pad - e
        row = jnp.concatenate([row, jnp.zeros((pad,), jnp.int32)])
        col = jnp.concatenate([col, jnp.zeros((pad,), jnp.int32)])
        val = jnp.concatenate([val, jnp.zeros((pad,), jnp.float32)])
    nb = e_pad // (NS * B)
    assert nb == NB, (nb, NB)
    col3 = col.reshape(NS, NB, B)
    row3 = row.reshape(NS, NB, B)
    val3 = val.reshape(NS, NB, B)
    xp = jnp.pad(x, ((0, NP - N), (0, 0)))
    xh = jnp.stack([xp[:, :DH], xp[:, DH:]])
    partial = _sc_spmv_partial(xh, col3, row3, val3)
    return _tc_concat(partial)[:N]


# speedup vs baseline: 9.6790x; 1.0935x over previous
import jax
import jax.numpy as jnp
from jax import lax
from jax.experimental import pallas as pl
from jax.experimental.pallas import tpu as pltpu
from jax.experimental.pallas import tpu_sc as plsc

N = 10000
NP = 10240
D = 128
DH = D // 2
NC = 2
NS = 16
B = 128
PASS = 40
NPASS = 4
NB = PASS * NPASS
ROWS_PER_TILE = NP // NS


def _bcast_lane(v16, lane):
    idx = jnp.full((16, 1), lane, dtype=jnp.int32)
    dn = lax.GatherDimensionNumbers(
        offset_dims=(), collapsed_slice_dims=(0,), start_index_map=(0,))
    return lax.gather(v16, idx, dn, slice_sizes=(1,),
                      mode=lax.GatherScatterMode.PROMISE_IN_BOUNDS)


def _sc_spmv_partial(xh, col, row, val):
    mesh = plsc.VectorSubcoreMesh(core_axis_name="c", subcore_axis_name="s")

    def body(x_hbm, col_hbm, row_hbm, val_hbm, out_hbm,
             colbuf, rowbuf, valbuf,
             gbuf0, gbuf1, sbuf0, sbuf1, xsp, acc_sh,
             gsem0, gsem1, ssem0, ssem1):
        gbufs = (gbuf0, gbuf1)
        sbufs = (sbuf0, sbuf1)
        gsems = (gsem0, gsem1)
        ssems = (ssem0, ssem1)
        cid = lax.axis_index("c")
        sid = lax.axis_index("s")
        off = sid * ROWS_PER_TILE

        @pl.when(sid < NS - 1)
        def _():
            pltpu.sync_copy(
                x_hbm.at[pl.ds(off, ROWS_PER_TILE), pl.ds(cid * DH, DH)],
                xsp.at[pl.ds(off, ROWS_PER_TILE)])

        @pl.when(sid == NS - 1)
        def _():
            pltpu.sync_copy(
                x_hbm.at[pl.ds(off, N - (NS - 1) * ROWS_PER_TILE),
                         pl.ds(cid * DH, DH)],
                xsp.at[pl.ds(off, N - (NS - 1) * ROWS_PER_TILE)])

        def zbody(i, carry):
            for k in range(DH // 16):
                sbuf0[i, pl.ds(k * 16, 16)] = jnp.zeros((16,), jnp.float32)
            return carry
        lax.fori_loop(0, B, zbody, 0)
        for c in range(ROWS_PER_TILE // B):
            pltpu.sync_copy(sbuf0, acc_sh.at[pl.ds(off + c * B, B)])
        plsc.subcore_barrier()

        for ps in range(NPASS):
            sl = pl.ds(ps * PASS, PASS)
            pltpu.sync_copy(col_hbm.at[sid, sl], colbuf)
            pltpu.sync_copy(row_hbm.at[sid, sl], rowbuf)
            pltpu.sync_copy(val_hbm.at[sid, sl], valbuf)

            for p in range(2):
                pltpu.async_copy(xsp.at[colbuf.at[p]], gbufs[p], gsems[p])

            def outer(t, carry):
                for p in range(2):
                    i = 2 * t + p
                    gb, sb = gbufs[p], sbufs[p]
                    pltpu.make_async_copy(xsp.at[colbuf.at[i]], gb,
                                          gsems[p]).wait()

                    @pl.when(jnp.logical_or(t > 0, ps > 0))
                    def _():
                        pltpu.make_async_copy(
                            sb, acc_sh.at[rowbuf.at[i]], ssems[p]).wait()

                    def jbody(j, c2):
                        v16 = valbuf[i, pl.ds(j * 16, 16)]
                        for e in range(16):
                            bv = _bcast_lane(v16, e)
                            r = j * 16 + e
                            for k in range(DH // 16):
                                sb[r, pl.ds(k * 16, 16)] = (
                                    gb[r, pl.ds(k * 16, 16)] * bv)
                        return c2
                    lax.fori_loop(0, B // 16, jbody, 0)

                    pltpu.async_copy(sb, acc_sh.at[rowbuf.at[i]], ssems[p],
                                     add=True)

                    @pl.when(t < PASS // 2 - 1)
                    def _():
                        pltpu.async_copy(xsp.at[colbuf.at[i + 2]], gb,
                                         gsems[p])
                return carry
            lax.fori_loop(0, PASS // 2, outer, 0)

        for p in range(2):
            pltpu.make_async_copy(sbufs[p], acc_sh.at[rowbuf.at[PASS - 2 + p]],
                                  ssems[p]).wait()

        plsc.subcore_barrier()

        pltpu.sync_copy(acc_sh.at[pl.ds(off, ROWS_PER_TILE)],
                        out_hbm.at[cid, pl.ds(off, ROWS_PER_TILE)])

    run = pl.kernel(
        body,
        mesh=mesh,
        compiler_params=pltpu.CompilerParams(use_tc_tiling_on_sc=False),
        out_type=jax.ShapeDtypeStruct((NC, NP, DH), jnp.float32),
        scratch_types=[
            pltpu.VMEM((PASS, B), jnp.int32),
            pltpu.VMEM((PASS, B), jnp.int32),
            pltpu.VMEM((PASS, B), jnp.float32),
            pltpu.VMEM((B, DH), jnp.float32),
            pltpu.VMEM((B, DH), jnp.float32),
            pltpu.VMEM((B, DH), jnp.float32),
            pltpu.VMEM((B, DH), jnp.float32),
            pltpu.VMEM_SHARED((NP, DH), jnp.float32),
            pltpu.VMEM_SHARED((NP, DH), jnp.float32),
            pltpu.SemaphoreType.DMA,
            pltpu.SemaphoreType.DMA,
            pltpu.SemaphoreType.DMA,
            pltpu.SemaphoreType.DMA,
        ],
    )
    return run(xh, col, row, val)


def _tc_concat(partial):
    BLK = 1000

    def body(p_ref, o_ref):
        o_ref[:, :DH] = p_ref[0]
        o_ref[:, DH:] = p_ref[1]

    return pl.pallas_call(
        body,
        out_shape=jax.ShapeDtypeStruct((N, D), jnp.float32),
        grid=(N // BLK,),
        in_specs=[pl.BlockSpec((NC, BLK, DH), lambda i: (0, i, 0))],
        out_specs=pl.BlockSpec((BLK, D), lambda i: (i, 0)),
    )(partial)


@jax.jit
def kernel(x, A_ind, A_val):
    row = A_ind[0].astype(jnp.int32)
    col = A_ind[1].astype(jnp.int32)
    val = A_val.astype(jnp.float32)
    e = val.shape[0]
    chunk = NS * B * 2 * NPASS
    e_pad = ((e + chunk - 1) // chunk) * chunk
    if e_pad != e:
        pad = e_pad - e
        row = jnp.concatenate([row, jnp.zeros((pad,), jnp.int32)])
        col = jnp.concatenate([col, jnp.zeros((pad,), jnp.int32)])
        val = jnp.concatenate([val, jnp.zeros((pad,), jnp.float32)])
    nb = e_pad // (NS * B)
    assert nb == NB, (nb, NB)
    col3 = col.reshape(NS, NB, B)
    row3 = row.reshape(NS, NB, B)
    val3 = val.reshape(NS, NB, B)
    partial = _sc_spmv_partial(x, col3, row3, val3)
    return _tc_concat(partial)


# speedup vs baseline: 10.6181x; 1.0970x over previous
import jax
import jax.numpy as jnp
from jax import lax
from jax.experimental import pallas as pl
from jax.experimental.pallas import tpu as pltpu
from jax.experimental.pallas import tpu_sc as plsc

N = 10000
NP = 10240
D = 128
DH = D // 2
NC = 2
NS = 16
B = 128
PASS = 40
NPASS = 4
NB = PASS * NPASS
ROWS_PER_TILE = NP // NS


def _bcast_lane(v16, lane):
    idx = jnp.full((16, 1), lane, dtype=jnp.int32)
    dn = lax.GatherDimensionNumbers(
        offset_dims=(), collapsed_slice_dims=(0,), start_index_map=(0,))
    return lax.gather(v16, idx, dn, slice_sizes=(1,),
                      mode=lax.GatherScatterMode.PROMISE_IN_BOUNDS)


def _sc_spmv_partial(xh, col, row, val):
    mesh = plsc.VectorSubcoreMesh(core_axis_name="c", subcore_axis_name="s")

    def body(x_hbm, col_hbm, row_hbm, val_hbm, out_hbm,
             colbuf, rowbuf, valbuf,
             gbuf0, gbuf1, sbuf0, sbuf1, xsp, acc_sh,
             gsem0, gsem1, ssem0, ssem1):
        gbufs = (gbuf0, gbuf1)
        sbufs = (sbuf0, sbuf1)
        gsems = (gsem0, gsem1)
        ssems = (ssem0, ssem1)
        cid = lax.axis_index("c")
        sid = lax.axis_index("s")
        off = sid * ROWS_PER_TILE

        @pl.when(sid < NS - 1)
        def _():
            pltpu.sync_copy(
                x_hbm.at[pl.ds(off, ROWS_PER_TILE), pl.ds(cid * DH, DH)],
                xsp.at[pl.ds(off, ROWS_PER_TILE)])

        @pl.when(sid == NS - 1)
        def _():
            pltpu.sync_copy(
                x_hbm.at[pl.ds(off, N - (NS - 1) * ROWS_PER_TILE),
                         pl.ds(cid * DH, DH)],
                xsp.at[pl.ds(off, N - (NS - 1) * ROWS_PER_TILE)])

        def zbody(i, carry):
            for k in range(DH // 16):
                sbuf0[i, pl.ds(k * 16, 16)] = jnp.zeros((16,), jnp.float32)
            return carry
        lax.fori_loop(0, B, zbody, 0)
        for c in range(ROWS_PER_TILE // B):
            pltpu.sync_copy(sbuf0, acc_sh.at[pl.ds(off + c * B, B)])
        plsc.subcore_barrier()

        for ps in range(NPASS):
            sl = pl.ds(ps * PASS, PASS)
            pltpu.sync_copy(col_hbm.at[sid, sl], colbuf)
            pltpu.sync_copy(row_hbm.at[sid, sl], rowbuf)
            pltpu.sync_copy(val_hbm.at[sid, sl], valbuf)

            for p in range(2):
                pltpu.async_copy(xsp.at[colbuf.at[p]], gbufs[p], gsems[p])

            def outer(t, carry):
                for p in range(2):
                    i = 2 * t + p
                    gb, sb = gbufs[p], sbufs[p]
                    pltpu.make_async_copy(xsp.at[colbuf.at[i]], gb,
                                          gsems[p]).wait()

                    @pl.when(jnp.logical_or(t > 0, ps > 0))
                    def _():
                        pltpu.make_async_copy(
                            sb, acc_sh.at[rowbuf.at[i]], ssems[p]).wait()

                    def jbody(j, c2):
                        v16 = valbuf[i, pl.ds(j * 16, 16)]
                        for e in range(16):
                            bv = _bcast_lane(v16, e)
                            r = j * 16 + e
                            for k in range(DH // 16):
                                sb[r, pl.ds(k * 16, 16)] = (
                                    gb[r, pl.ds(k * 16, 16)] * bv)
                        return c2
                    lax.fori_loop(0, B // 16, jbody, 0)

                    pltpu.async_copy(sb, acc_sh.at[rowbuf.at[i]], ssems[p],
                                     add=True)

                    @pl.when(t < PASS // 2 - 1)
                    def _():
                        pltpu.async_copy(xsp.at[colbuf.at[i + 2]], gb,
                                         gsems[p])
                return carry
            lax.fori_loop(0, PASS // 2, outer, 0)

        for p in range(2):
            pltpu.make_async_copy(sbufs[p], acc_sh.at[rowbuf.at[PASS - 2 + p]],
                                  ssems[p]).wait()

        plsc.subcore_barrier()

        @pl.when(sid < NS - 1)
        def _():
            pltpu.sync_copy(
                acc_sh.at[pl.ds(off, ROWS_PER_TILE)],
                out_hbm.at[pl.ds(off, ROWS_PER_TILE), pl.ds(cid * DH, DH)])

        @pl.when(sid == NS - 1)
        def _():
            pltpu.sync_copy(
                acc_sh.at[pl.ds(off, N - (NS - 1) * ROWS_PER_TILE)],
                out_hbm.at[pl.ds(off, N - (NS - 1) * ROWS_PER_TILE),
                           pl.ds(cid * DH, DH)])

    run = pl.kernel(
        body,
        mesh=mesh,
        compiler_params=pltpu.CompilerParams(use_tc_tiling_on_sc=False),
        out_type=jax.ShapeDtypeStruct((N, D), jnp.float32),
        scratch_types=[
            pltpu.VMEM((PASS, B), jnp.int32),
            pltpu.VMEM((PASS, B), jnp.int32),
            pltpu.VMEM((PASS, B), jnp.float32),
            pltpu.VMEM((B, DH), jnp.float32),
            pltpu.VMEM((B, DH), jnp.float32),
            pltpu.VMEM((B, DH), jnp.float32),
            pltpu.VMEM((B, DH), jnp.float32),
            pltpu.VMEM_SHARED((NP, DH), jnp.float32),
            pltpu.VMEM_SHARED((NP, DH), jnp.float32),
            pltpu.SemaphoreType.DMA,
            pltpu.SemaphoreType.DMA,
            pltpu.SemaphoreType.DMA,
            pltpu.SemaphoreType.DMA,
        ],
    )
    return run(xh, col, row, val)


def _tc_pad_edges(row, col, val, e_pad):
    e = val.shape[0]

    def body(r_ref, c_ref, v_ref, ro_ref, co_ref, vo_ref):
        for src, dst in ((r_ref, ro_ref), (c_ref, co_ref), (v_ref, vo_ref)):
            dst[pl.ds(0, e)] = src[...]
            dst[pl.ds(e, e_pad - e)] = jnp.zeros((e_pad - e,), src.dtype)

    return pl.pallas_call(
        body,
        out_shape=(jax.ShapeDtypeStruct((e_pad,), jnp.int32),
                   jax.ShapeDtypeStruct((e_pad,), jnp.int32),
                   jax.ShapeDtypeStruct((e_pad,), jnp.float32)),
    )(row, col, val)


@jax.jit
def kernel(x, A_ind, A_val):
    row = A_ind[0].astype(jnp.int32)
    col = A_ind[1].astype(jnp.int32)
    val = A_val.astype(jnp.float32)
    e = val.shape[0]
    chunk = NS * B * 2 * NPASS
    e_pad = ((e + chunk - 1) // chunk) * chunk
    if e_pad != e:
        row, col, val = _tc_pad_edges(row, col, val, e_pad)
    nb = e_pad // (NS * B)
    assert nb == NB, (nb, NB)
    col3 = col.reshape(NS, NB, B)
    row3 = row.reshape(NS, NB, B)
    val3 = val.reshape(NS, NB, B)
    return _sc_spmv_partial(x, col3, row3, val3)
